# baseline (speedup 1.0000x reference)
import jax
import jax.numpy as jnp
from jax.experimental import pallas as pl
from jax.experimental.pallas import tpu as pltpu


def _se_twophase_kernel(x_ref, w1_ref, b1_ref, w2_ref, b2_ref, o_ref,
                        stash_ref, acc_ref, gate_ref, *, hw, t, n_t):
    p = pl.program_id(1)
    j = pl.program_id(2)

    @pl.when((p == 0) & (j == 0))
    def _init():
        acc_ref[...] = jnp.zeros_like(acc_ref)

    @pl.when(p == 0)
    def _read_phase():
        x = x_ref[...]
        stash_ref[:, :, pl.ds(j * t, t)] = x
        if hw % t != 0:
            lane = jax.lax.broadcasted_iota(jnp.int32, x.shape, dimension=2)
            x = jnp.where(j * t + lane < hw, x, jnp.zeros_like(x))
        acc_ref[...] += jnp.sum(x, axis=-1, dtype=jnp.float32)

    @pl.when((p == 0) & (j == n_t - 1))
    def _mlp():
        y = acc_ref[...] / jnp.float32(hw)
        h = jnp.dot(y, w1_ref[...], preferred_element_type=jnp.float32)
        h = jnp.maximum(h + b1_ref[...], 0.0)
        g = jnp.dot(h, w2_ref[...], preferred_element_type=jnp.float32)
        gate_ref[...] = jax.nn.sigmoid(g + b2_ref[...])

    @pl.when(p == 1)
    def _write_phase():
        xs = stash_ref[:, :, pl.ds(j * t, t)]
        o_ref[...] = xs * gate_ref[...][:, :, None]


def kernel(x, w1, b1, w2, b2):
    B, C, H, W = x.shape
    HW = H * W
    Cr = w1.shape[0]

    x3 = x.reshape(B, C, HW)
    w1_t = w1.T
    w2_t = w2.T
    b1r = b1.reshape(1, Cr)
    b2r = b2.reshape(1, C)

    BB = 4
    n_b = B // BB
    t = 768
    n_t = pl.cdiv(HW, t)

    itemsize = jnp.dtype(x3.dtype).itemsize
    x_bytes = B * C * HW * itemsize

    out3 = pl.pallas_call(
        lambda *refs: _se_twophase_kernel(*refs, hw=HW, t=t, n_t=n_t),
        out_shape=jax.ShapeDtypeStruct((B, C, HW), x3.dtype),
        grid=(n_b, 2, n_t),
        in_specs=[
            pl.BlockSpec((BB, C, t),
                         lambda b, p, j: (b, 0, (1 - p) * j + p * (n_t - 1))),
            pl.BlockSpec((C, Cr), lambda b, p, j: (0, 0)),
            pl.BlockSpec((1, Cr), lambda b, p, j: (0, 0)),
            pl.BlockSpec((Cr, C), lambda b, p, j: (0, 0)),
            pl.BlockSpec((1, C), lambda b, p, j: (0, 0)),
        ],
        out_specs=pl.BlockSpec((BB, C, t), lambda b, p, j: (b, 0, p * j)),
        scratch_shapes=[
            pltpu.VMEM((BB, C, n_t * t), x3.dtype),
            pltpu.VMEM((BB, C), jnp.float32),
            pltpu.VMEM((BB, C), jnp.float32),
        ],
        compiler_params=pltpu.CompilerParams(
            dimension_semantics=("parallel", "arbitrary", "arbitrary"),
            vmem_limit_bytes=60 << 20,
        ),
        cost_estimate=pl.CostEstimate(
            flops=2 * B * C * HW + 4 * B * C * Cr,
            transcendentals=B * C,
            bytes_accessed=2 * x_bytes,
        ),
    )(x3, w1_t, b1r, w2_t, b2r)
    return out3.reshape(B, C, H, W)


# speedup vs baseline: 1.1139x; 1.1139x over previous
import jax
import jax.numpy as jnp
from jax.experimental import pallas as pl
from jax.experimental.pallas import tpu as pltpu


def _se_onepass_kernel(x_ref, w1_ref, b1_ref, w2_ref, b2_ref, o_ref):
    x = x_ref[...]
    inv_hw = jnp.float32(1.0 / x.shape[-1])
    y = jnp.sum(x, axis=-1, dtype=jnp.float32) * inv_hw
    h = jnp.dot(y, w1_ref[...], preferred_element_type=jnp.float32)
    h = jnp.maximum(h + b1_ref[...], 0.0)
    g = jnp.dot(h, w2_ref[...], preferred_element_type=jnp.float32)
    g = jax.nn.sigmoid(g + b2_ref[...])
    o_ref[...] = x * g.astype(x.dtype)[:, :, None]


def kernel(x, w1, b1, w2, b2):
    B, C, H, W = x.shape
    HW = H * W
    Cr = w1.shape[0]

    x3 = x.reshape(B, C, HW)
    w1_t = w1.T
    w2_t = w2.T
    b1r = b1.reshape(1, Cr)
    b2r = b2.reshape(1, C)

    BB = 4
    n_b = B // BB

    itemsize = jnp.dtype(x3.dtype).itemsize
    x_bytes = B * C * HW * itemsize

    out3 = pl.pallas_call(
        _se_onepass_kernel,
        out_shape=jax.ShapeDtypeStruct((B, C, HW), x3.dtype),
        grid=(n_b,),
        in_specs=[
            pl.BlockSpec((BB, C, HW), lambda b: (b, 0, 0)),
            pl.BlockSpec((C, Cr), lambda b: (0, 0)),
            pl.BlockSpec((1, Cr), lambda b: (0, 0)),
            pl.BlockSpec((Cr, C), lambda b: (0, 0)),
            pl.BlockSpec((1, C), lambda b: (0, 0)),
        ],
        out_specs=pl.BlockSpec((BB, C, HW), lambda b: (b, 0, 0)),
        compiler_params=pltpu.CompilerParams(
            dimension_semantics=("parallel",),
            vmem_limit_bytes=60 << 20,
        ),
        cost_estimate=pl.CostEstimate(
            flops=2 * B * C * HW + 4 * B * C * Cr,
            transcendentals=B * C,
            bytes_accessed=2 * x_bytes,
        ),
    )(x3, w1_t, b1r, w2_t, b2r)
    return out3.reshape(B, C, H, W)
